# baseline (speedup 1.0000x reference)
import jax
import jax.numpy as jnp
from jax.experimental import pallas as pl
from jax.experimental.pallas import tpu as pltpu

_NCHUNK = 8


def _swap_body(x_hbm, o_hbm, sem):
    n = x_hbm.shape[0]
    half = n // 2
    ch = half // _NCHUNK
    copies = []
    for k in range(_NCHUNK):
        copies.append(pltpu.make_async_copy(
            x_hbm.at[pl.ds(half + k * ch, ch)], o_hbm.at[pl.ds(k * ch, ch)], sem))
        copies.append(pltpu.make_async_copy(
            x_hbm.at[pl.ds(k * ch, ch)], o_hbm.at[pl.ds(half + k * ch, ch)], sem))
    for cp in copies:
        cp.start()
    for cp in copies:
        cp.wait()


def kernel(x):
    return pl.pallas_call(
        _swap_body,
        in_specs=[pl.BlockSpec(memory_space=pltpu.MemorySpace.HBM)],
        out_specs=pl.BlockSpec(memory_space=pltpu.MemorySpace.HBM),
        out_shape=jax.ShapeDtypeStruct(x.shape, x.dtype),
        scratch_shapes=[pltpu.SemaphoreType.DMA],
    )(x)


# speedup vs baseline: 18.0725x; 18.0725x over previous
import jax
import jax.numpy as jnp
from jax.experimental import pallas as pl


def _copy_body(x_ref, o_ref):
    o_ref[...] = x_ref[...]


def kernel(x):
    n, c = x.shape
    br = 16384
    nb = n // br
    return pl.pallas_call(
        _copy_body,
        grid=(nb,),
        in_specs=[pl.BlockSpec((br, c), lambda i: ((i + nb // 2) % nb, 0))],
        out_specs=pl.BlockSpec((br, c), lambda i: (i, 0)),
        out_shape=jax.ShapeDtypeStruct(x.shape, x.dtype),
    )(x)
